# baseline (speedup 1.0000x reference)
import functools

import jax
import jax.numpy as jnp
from jax import lax
from jax.experimental import pallas as pl
from jax.experimental.pallas import tpu as pltpu
from jax.experimental.pallas import tpu_sc as plsc

NPAD = 10240
EPAD = 163840
NC, NS = 2, 16
NT = NC * NS
EPT = EPAD // NT
CH = 128
NCHUNK = EPT // CH
NCHP = EPAD // NS // CH
RPT = NPAD // NS
DW = 128
RB = 512
GRID = NPAD // RB

_mesh = plsc.VectorSubcoreMesh(core_axis_name="c", subcore_axis_name="s")


def _deg_body(dst_hbm, ones_hbm, zero_hbm, out_hbm, idxd0, idxd1, ones_v,
              acc_sh, isem0, isem1):
    c = lax.axis_index("c")
    s = lax.axis_index("s")
    wid = c * NS + s
    base = s * RPT
    ebase = wid * EPT

    pltpu.sync_copy(ones_hbm, ones_v)
    pltpu.sync_copy(zero_hbm, acc_sh.at[pl.ds(base, RPT)])

    pltpu.async_copy(dst_hbm.at[pl.ds(ebase, CH)], idxd0, isem0)
    plsc.subcore_barrier()

    def body(g, _):
        j0 = 2 * g
        pltpu.async_copy(dst_hbm.at[pl.ds(ebase + (j0 + 1) * CH, CH)],
                         idxd1, isem1)
        pltpu.make_async_copy(dst_hbm.at[pl.ds(0, CH)], idxd0, isem0).wait()
        pltpu.sync_copy(ones_v, acc_sh.at[idxd0], add=True)

        @pl.when(j0 + 2 < NCHUNK)
        def _():
            pltpu.async_copy(dst_hbm.at[pl.ds(ebase + (j0 + 2) * CH, CH)],
                             idxd0, isem0)

        pltpu.make_async_copy(dst_hbm.at[pl.ds(0, CH)], idxd1, isem1).wait()
        pltpu.sync_copy(ones_v, acc_sh.at[idxd1], add=True)
        return 0

    lax.fori_loop(0, NCHUNK // 2, body, 0)
    plsc.subcore_barrier()
    pltpu.sync_copy(acc_sh.at[pl.ds(base, RPT)],
                    out_hbm.at[pl.ds(c * NPAD + base, RPT)])


_sc_deg = functools.partial(
    pl.kernel,
    out_type=jax.ShapeDtypeStruct((NC * NPAD, DW), jnp.float32),
    mesh=_mesh,
    scratch_types=[
        pltpu.VMEM((CH,), jnp.int32),
        pltpu.VMEM((CH,), jnp.int32),
        pltpu.VMEM((CH, DW), jnp.float32),
        pltpu.VMEM_SHARED((NPAD, DW), jnp.float32),
        pltpu.SemaphoreType.DMA,
        pltpu.SemaphoreType.DMA,
    ],
)(_deg_body)


def _prop_body(hs_hbm, src_hbm, dst_hbm, zero_hbm, out_hbm,
               idxs0, idxs1, idxd0, idxd1, rows0, rows1,
               acc_sh, ssem0, ssem1, dsem0, dsem1, gsem0, gsem1):
    c = lax.axis_index("c")
    s = lax.axis_index("s")
    base = s * RPT
    nch = NCHP
    ebase = s * NCHP * CH

    @pl.when(c == 0)
    def _():
        pltpu.sync_copy(zero_hbm, acc_sh.at[pl.ds(base, RPT)])
        pltpu.async_copy(src_hbm.at[pl.ds(ebase, CH)], idxs0, ssem0)
        pltpu.async_copy(dst_hbm.at[pl.ds(ebase, CH)], idxd0, dsem0)
        pltpu.make_async_copy(src_hbm.at[pl.ds(0, CH)], idxs0, ssem0).wait()
        pltpu.async_copy(hs_hbm.at[idxs0], rows0, gsem0)

    plsc.subcore_barrier()

    def body(g, _):
        j0 = 2 * g

        pltpu.async_copy(src_hbm.at[pl.ds(ebase + (j0 + 1) * CH, CH)],
                         idxs1, ssem1)
        pltpu.async_copy(dst_hbm.at[pl.ds(ebase + (j0 + 1) * CH, CH)],
                         idxd1, dsem1)
        pltpu.make_async_copy(src_hbm.at[pl.ds(0, CH)], idxs1, ssem1).wait()
        pltpu.async_copy(hs_hbm.at[idxs1], rows1, gsem1)

        pltpu.make_async_copy(hs_hbm.at[idxs0], rows0, gsem0).wait()
        pltpu.make_async_copy(dst_hbm.at[pl.ds(0, CH)], idxd0, dsem0).wait()
        pltpu.sync_copy(rows0, acc_sh.at[idxd0], add=True)

        @pl.when(j0 + 2 < nch)
        def _():
            pltpu.async_copy(src_hbm.at[pl.ds(ebase + (j0 + 2) * CH, CH)],
                             idxs0, ssem0)
            pltpu.async_copy(dst_hbm.at[pl.ds(ebase + (j0 + 2) * CH, CH)],
                             idxd0, dsem0)
            pltpu.make_async_copy(src_hbm.at[pl.ds(0, CH)], idxs0, ssem0).wait()
            pltpu.async_copy(hs_hbm.at[idxs0], rows0, gsem0)

        pltpu.make_async_copy(hs_hbm.at[idxs1], rows1, gsem1).wait()
        pltpu.make_async_copy(dst_hbm.at[pl.ds(0, CH)], idxd1, dsem1).wait()
        pltpu.sync_copy(rows1, acc_sh.at[idxd1], add=True)
        return 0

    @pl.when(c == 0)
    def _():
        lax.fori_loop(0, nch // 2, body, 0)

    plsc.subcore_barrier()

    @pl.when(c == 0)
    def _():
        pltpu.sync_copy(acc_sh.at[pl.ds(base, RPT)],
                        out_hbm.at[pl.ds(base, RPT)])


_sc_prop = functools.partial(
    pl.kernel,
    out_type=jax.ShapeDtypeStruct((NPAD, 128), jnp.float32),
    mesh=_mesh,
    scratch_types=[
        pltpu.VMEM((CH,), jnp.int32),
        pltpu.VMEM((CH,), jnp.int32),
        pltpu.VMEM((CH,), jnp.int32),
        pltpu.VMEM((CH,), jnp.int32),
        pltpu.VMEM((CH, 128), jnp.float32),
        pltpu.VMEM((CH, 128), jnp.float32),
        pltpu.VMEM_SHARED((NPAD, 128), jnp.float32),
        pltpu.SemaphoreType.DMA,
        pltpu.SemaphoreType.DMA,
        pltpu.SemaphoreType.DMA,
        pltpu.SemaphoreType.DMA,
        pltpu.SemaphoreType.DMA,
        pltpu.SemaphoreType.DMA,
    ],
)(_prop_body)


def _dinv_from(degp_ref):
    deg = degp_ref[0][:, 0:1] + degp_ref[1][:, 0:1] + 1.0
    return lax.rsqrt(deg)


def _tc_in_body(x_ref, w_ref, degp_ref, o_ref):
    dinv = _dinv_from(degp_ref)
    o_ref[...] = jnp.dot(x_ref[...], w_ref[...],
                         preferred_element_type=jnp.float32) * dinv


def _tc_mid_body(acc_ref, hs_ref, degp_ref, b_ref, o_ref):
    dinv = _dinv_from(degp_ref)
    z = jnp.maximum(dinv * (acc_ref[...] + hs_ref[...]) + b_ref[...], 0.0)
    o_ref[...] = dinv * z


def _tc_out_body(acc_ref, hs_ref, degp_ref, w_ref, b_ref, o_ref):
    dinv = _dinv_from(degp_ref)
    p = dinv * (acc_ref[...] + hs_ref[...])
    o_ref[...] = jnp.dot(p, w_ref[...],
                         preferred_element_type=jnp.float32) + b_ref[...]


def _tc_in(xp, W1, degp):
    return pl.pallas_call(
        _tc_in_body,
        grid=(GRID,),
        in_specs=[
            pl.BlockSpec((RB, 256), lambda i: (i, 0)),
            pl.BlockSpec((256, 128), lambda i: (0, 0)),
            pl.BlockSpec((2, RB, DW), lambda i: (0, i, 0)),
        ],
        out_specs=pl.BlockSpec((RB, 128), lambda i: (i, 0)),
        out_shape=jax.ShapeDtypeStruct((NPAD, 128), jnp.float32),
    )(xp, W1, degp)


def _tc_mid(acc, hs, degp, b1):
    return pl.pallas_call(
        _tc_mid_body,
        grid=(GRID,),
        in_specs=[
            pl.BlockSpec((RB, 128), lambda i: (i, 0)),
            pl.BlockSpec((RB, 128), lambda i: (i, 0)),
            pl.BlockSpec((2, RB, DW), lambda i: (0, i, 0)),
            pl.BlockSpec((1, 128), lambda i: (0, 0)),
        ],
        out_specs=pl.BlockSpec((RB, 128), lambda i: (i, 0)),
        out_shape=jax.ShapeDtypeStruct((NPAD, 128), jnp.float32),
    )(acc, hs, degp, b1)


def _tc_out(acc, hs, degp, W2, b2):
    return pl.pallas_call(
        _tc_out_body,
        grid=(GRID,),
        in_specs=[
            pl.BlockSpec((RB, 128), lambda i: (i, 0)),
            pl.BlockSpec((RB, 128), lambda i: (i, 0)),
            pl.BlockSpec((2, RB, DW), lambda i: (0, i, 0)),
            pl.BlockSpec((128, 256), lambda i: (0, 0)),
            pl.BlockSpec((1, 256), lambda i: (0, 0)),
        ],
        out_specs=pl.BlockSpec((RB, 256), lambda i: (i, 0)),
        out_shape=jax.ShapeDtypeStruct((NPAD, 256), jnp.float32),
    )(acc, hs, degp, W2, b2)


def kernel(x, edge_index, W1, b1, W2, b2):
    n = x.shape[0]
    e = edge_index.shape[1]
    pad_idx = jnp.full((EPAD - e,), NPAD - 1, jnp.int32)
    srcf = jnp.concatenate([edge_index[0].astype(jnp.int32), pad_idx])
    dstf = jnp.concatenate([edge_index[1].astype(jnp.int32), pad_idx])
    xp = jnp.pad(x, ((0, NPAD - n), (0, 0)))

    ones_rows = jnp.ones((CH, DW), jnp.float32)
    zero_rows = jnp.zeros((RPT, 128), jnp.float32)

    degp = _sc_deg(dstf, ones_rows, zero_rows[:, :DW]).reshape(NC, NPAD, DW)
    hs1 = _tc_in(xp, W1, degp)
    acc1 = _sc_prop(hs1, srcf, dstf, zero_rows)
    hs2 = _tc_mid(acc1, hs1, degp, b1.reshape(1, 128))
    acc2 = _sc_prop(hs2, srcf, dstf, zero_rows)
    outp = _tc_out(acc2, hs2, degp, W2, b2.reshape(1, 256))
    return outp[:n]


# speedup vs baseline: 1.2144x; 1.2144x over previous
import functools

import jax
import jax.numpy as jnp
from jax import lax
from jax.experimental import pallas as pl
from jax.experimental.pallas import tpu as pltpu
from jax.experimental.pallas import tpu_sc as plsc

NPAD = 10240
EPAD = 163840
NC, NS = 2, 16
NT = NC * NS
EPT = EPAD // NT
CH = 128
NCHUNK = EPT // CH
NCH0 = 60
NCH1 = 20
RPT = NPAD // NS
DW = 128
RB = 512
GRID = NPAD // RB

_mesh = plsc.VectorSubcoreMesh(core_axis_name="c", subcore_axis_name="s")


def _deg_body(dst_hbm, ones_hbm, zero_hbm, out_hbm, idxd0, idxd1, ones_v,
              acc_sh, isem0, isem1):
    c = lax.axis_index("c")
    s = lax.axis_index("s")
    wid = c * NS + s
    base = s * RPT
    ebase = wid * EPT

    pltpu.sync_copy(ones_hbm, ones_v)
    pltpu.sync_copy(zero_hbm, acc_sh.at[pl.ds(base, RPT)])

    pltpu.async_copy(dst_hbm.at[pl.ds(ebase, CH)], idxd0, isem0)
    plsc.subcore_barrier()

    def body(g, _):
        j0 = 2 * g
        pltpu.async_copy(dst_hbm.at[pl.ds(ebase + (j0 + 1) * CH, CH)],
                         idxd1, isem1)
        pltpu.make_async_copy(dst_hbm.at[pl.ds(0, CH)], idxd0, isem0).wait()
        pltpu.sync_copy(ones_v, acc_sh.at[idxd0], add=True)

        @pl.when(j0 + 2 < NCHUNK)
        def _():
            pltpu.async_copy(dst_hbm.at[pl.ds(ebase + (j0 + 2) * CH, CH)],
                             idxd0, isem0)

        pltpu.make_async_copy(dst_hbm.at[pl.ds(0, CH)], idxd1, isem1).wait()
        pltpu.sync_copy(ones_v, acc_sh.at[idxd1], add=True)
        return 0

    lax.fori_loop(0, NCHUNK // 2, body, 0)
    plsc.subcore_barrier()
    pltpu.sync_copy(acc_sh.at[pl.ds(base, RPT)],
                    out_hbm.at[pl.ds(c * NPAD + base, RPT)])


_sc_deg = functools.partial(
    pl.kernel,
    out_type=jax.ShapeDtypeStruct((NC * NPAD, DW), jnp.float32),
    mesh=_mesh,
    scratch_types=[
        pltpu.VMEM((CH,), jnp.int32),
        pltpu.VMEM((CH,), jnp.int32),
        pltpu.VMEM((CH, DW), jnp.float32),
        pltpu.VMEM_SHARED((NPAD, DW), jnp.float32),
        pltpu.SemaphoreType.DMA,
        pltpu.SemaphoreType.DMA,
    ],
)(_deg_body)


def _prop_body(hs_hbm, src_hbm, dst_hbm, zero_hbm, out_hbm,
               idxs0, idxs1, idxd0, idxd1, rows0, rows1,
               acc_sh, ssem0, ssem1, dsem0, dsem1, gsem0, gsem1):
    c = lax.axis_index("c")
    s = lax.axis_index("s")
    base = s * RPT
    nch = jnp.where(c == 0, NCH0, NCH1)
    ebase = jnp.where(c == 0, s * NCH0, NS * NCH0 + s * NCH1) * CH

    pltpu.sync_copy(zero_hbm, acc_sh.at[pl.ds(base, RPT)])

    pltpu.async_copy(src_hbm.at[pl.ds(ebase, CH)], idxs0, ssem0)
    pltpu.async_copy(dst_hbm.at[pl.ds(ebase, CH)], idxd0, dsem0)
    pltpu.make_async_copy(src_hbm.at[pl.ds(0, CH)], idxs0, ssem0).wait()
    pltpu.async_copy(hs_hbm.at[idxs0], rows0, gsem0)
    plsc.subcore_barrier()

    def body(g, _):
        j0 = 2 * g

        pltpu.async_copy(src_hbm.at[pl.ds(ebase + (j0 + 1) * CH, CH)],
                         idxs1, ssem1)
        pltpu.async_copy(dst_hbm.at[pl.ds(ebase + (j0 + 1) * CH, CH)],
                         idxd1, dsem1)
        pltpu.make_async_copy(src_hbm.at[pl.ds(0, CH)], idxs1, ssem1).wait()
        pltpu.async_copy(hs_hbm.at[idxs1], rows1, gsem1)

        pltpu.make_async_copy(hs_hbm.at[idxs0], rows0, gsem0).wait()
        pltpu.make_async_copy(dst_hbm.at[pl.ds(0, CH)], idxd0, dsem0).wait()
        pltpu.sync_copy(rows0, acc_sh.at[idxd0], add=True)

        @pl.when(j0 + 2 < nch)
        def _():
            pltpu.async_copy(src_hbm.at[pl.ds(ebase + (j0 + 2) * CH, CH)],
                             idxs0, ssem0)
            pltpu.async_copy(dst_hbm.at[pl.ds(ebase + (j0 + 2) * CH, CH)],
                             idxd0, dsem0)
            pltpu.make_async_copy(src_hbm.at[pl.ds(0, CH)], idxs0, ssem0).wait()
            pltpu.async_copy(hs_hbm.at[idxs0], rows0, gsem0)

        pltpu.make_async_copy(hs_hbm.at[idxs1], rows1, gsem1).wait()
        pltpu.make_async_copy(dst_hbm.at[pl.ds(0, CH)], idxd1, dsem1).wait()
        pltpu.sync_copy(rows1, acc_sh.at[idxd1], add=True)
        return 0

    lax.fori_loop(0, nch // 2, body, 0)
    plsc.subcore_barrier()
    pltpu.sync_copy(acc_sh.at[pl.ds(base, RPT)],
                    out_hbm.at[pl.ds(c * NPAD + base, RPT)])


_sc_prop = functools.partial(
    pl.kernel,
    out_type=jax.ShapeDtypeStruct((NC * NPAD, 128), jnp.float32),
    mesh=_mesh,
    scratch_types=[
        pltpu.VMEM((CH,), jnp.int32),
        pltpu.VMEM((CH,), jnp.int32),
        pltpu.VMEM((CH,), jnp.int32),
        pltpu.VMEM((CH,), jnp.int32),
        pltpu.VMEM((CH, 128), jnp.float32),
        pltpu.VMEM((CH, 128), jnp.float32),
        pltpu.VMEM_SHARED((NPAD, 128), jnp.float32),
        pltpu.SemaphoreType.DMA,
        pltpu.SemaphoreType.DMA,
        pltpu.SemaphoreType.DMA,
        pltpu.SemaphoreType.DMA,
        pltpu.SemaphoreType.DMA,
        pltpu.SemaphoreType.DMA,
    ],
)(_prop_body)


def _dinv_from(degp_ref):
    deg = degp_ref[0][:, 0:1] + degp_ref[1][:, 0:1] + 1.0
    return lax.rsqrt(deg)


def _tc_in_body(x_ref, w_ref, degp_ref, o_ref):
    dinv = _dinv_from(degp_ref)
    o_ref[...] = jnp.dot(x_ref[...], w_ref[...],
                         preferred_element_type=jnp.float32) * dinv


def _tc_mid_body(acc_ref, hs_ref, degp_ref, b_ref, o_ref):
    dinv = _dinv_from(degp_ref)
    z = jnp.maximum(dinv * (acc_ref[0] + acc_ref[1] + hs_ref[...]) + b_ref[...],
                    0.0)
    o_ref[...] = dinv * z


def _tc_out_body(acc_ref, hs_ref, degp_ref, w_ref, b_ref, o_ref):
    dinv = _dinv_from(degp_ref)
    p = dinv * (acc_ref[0] + acc_ref[1] + hs_ref[...])
    o_ref[...] = jnp.dot(p, w_ref[...],
                         preferred_element_type=jnp.float32) + b_ref[...]


def _tc_in(xp, W1, degp):
    return pl.pallas_call(
        _tc_in_body,
        grid=(GRID,),
        in_specs=[
            pl.BlockSpec((RB, 256), lambda i: (i, 0)),
            pl.BlockSpec((256, 128), lambda i: (0, 0)),
            pl.BlockSpec((2, RB, DW), lambda i: (0, i, 0)),
        ],
        out_specs=pl.BlockSpec((RB, 128), lambda i: (i, 0)),
        out_shape=jax.ShapeDtypeStruct((NPAD, 128), jnp.float32),
    )(xp, W1, degp)


def _tc_mid(acc, hs, degp, b1):
    return pl.pallas_call(
        _tc_mid_body,
        grid=(GRID,),
        in_specs=[
            pl.BlockSpec((2, RB, 128), lambda i: (0, i, 0)),
            pl.BlockSpec((RB, 128), lambda i: (i, 0)),
            pl.BlockSpec((2, RB, DW), lambda i: (0, i, 0)),
            pl.BlockSpec((1, 128), lambda i: (0, 0)),
        ],
        out_specs=pl.BlockSpec((RB, 128), lambda i: (i, 0)),
        out_shape=jax.ShapeDtypeStruct((NPAD, 128), jnp.float32),
    )(acc, hs, degp, b1)


def _tc_out(acc, hs, degp, W2, b2):
    return pl.pallas_call(
        _tc_out_body,
        grid=(GRID,),
        in_specs=[
            pl.BlockSpec((2, RB, 128), lambda i: (0, i, 0)),
            pl.BlockSpec((RB, 128), lambda i: (i, 0)),
            pl.BlockSpec((2, RB, DW), lambda i: (0, i, 0)),
            pl.BlockSpec((128, 256), lambda i: (0, 0)),
            pl.BlockSpec((1, 256), lambda i: (0, 0)),
        ],
        out_specs=pl.BlockSpec((RB, 256), lambda i: (i, 0)),
        out_shape=jax.ShapeDtypeStruct((NPAD, 256), jnp.float32),
    )(acc, hs, degp, W2, b2)


def kernel(x, edge_index, W1, b1, W2, b2):
    n = x.shape[0]
    e = edge_index.shape[1]
    pad_idx = jnp.full((EPAD - e,), NPAD - 1, jnp.int32)
    srcf = jnp.concatenate([edge_index[0].astype(jnp.int32), pad_idx])
    dstf = jnp.concatenate([edge_index[1].astype(jnp.int32), pad_idx])
    xp = jnp.pad(x, ((0, NPAD - n), (0, 0)))

    ones_rows = jnp.ones((CH, DW), jnp.float32)
    zero_rows = jnp.zeros((RPT, 128), jnp.float32)

    degp = _sc_deg(dstf, ones_rows, zero_rows[:, :DW]).reshape(NC, NPAD, DW)
    hs1 = _tc_in(xp, W1, degp)
    acc1 = _sc_prop(hs1, srcf, dstf, zero_rows).reshape(NC, NPAD, 128)
    hs2 = _tc_mid(acc1, hs1, degp, b1.reshape(1, 128))
    acc2 = _sc_prop(hs2, srcf, dstf, zero_rows).reshape(NC, NPAD, 128)
    outp = _tc_out(acc2, hs2, degp, W2, b2.reshape(1, 256))
    return outp[:n]
